# baseline (speedup 1.0000x reference)
import functools

import jax
import jax.numpy as jnp
from jax import lax
from jax.experimental import pallas as pl
from jax.experimental.pallas import tpu as pltpu
from jax.experimental.pallas import tpu_sc as plsc

N = 10000
D = 128
E = 320000
NC = 2
NS = 16
NW = NC * NS
EPW = E // NW
CH = 80
NCH = EPW // CH
NP = 10240
RPT = NP // NS
NWB = RPT // CH
NBUF = 4
NBI = 4

BN = 1000
GRID = N // BN


def _sc_mesh():
    return plsc.VectorSubcoreMesh(
        core_axis_name="c", subcore_axis_name="s",
        num_cores=NC, num_subcores=NS)


DW = 16


def _deg_partials(dst, ones16, zeros16):
    @functools.partial(
        pl.kernel,
        out_type=jax.ShapeDtypeStruct((NC * NP, DW), jnp.float32),
        mesh=_sc_mesh(),
        compiler_params=pltpu.CompilerParams(use_tc_tiling_on_sc=False),
        scratch_types=[
            pltpu.VMEM((NBI, CH), jnp.int32),
            pltpu.VMEM((CH, DW), jnp.float32),
            pltpu.VMEM((CH, DW), jnp.float32),
            pltpu.VMEM((CH, DW), jnp.float32),
            pltpu.VMEM_SHARED((NP, DW), jnp.float32),
            pltpu.SemaphoreType.DMA,
            pltpu.SemaphoreType.DMA,
            pltpu.SemaphoreType.DMA,
        ],
    )
    def k(dst_hbm, ones_hbm, zeros_hbm, out_hbm,
          idst, ones_v, wb0, wb1, acc, isem, ssem, wsem):
        c = lax.axis_index("c")
        s = lax.axis_index("s")
        wid = c * NS + s
        wbs = [wb0, wb1]

        def zslice(t):
            return acc.at[pl.ds(s * RPT + t * CH, CH)]

        def oslice(t):
            return out_hbm.at[pl.ds(c * NP + s * RPT + t * CH, CH)]

        def idx_issue(j):
            base = pl.multiple_of(wid * EPW + j * CH, 8)
            pltpu.async_copy(dst_hbm.at[pl.ds(base, CH)],
                             idst.at[lax.rem(j, NBI)], isem)

        def idx_wait(j):
            base = pl.multiple_of(wid * EPW + j * CH, 8)
            pltpu.make_async_copy(dst_hbm.at[pl.ds(base, CH)],
                                  idst.at[lax.rem(j, NBI)], isem).wait()

        def sc_wait():
            pltpu.make_async_copy(ones_v, acc.at[pl.ds(0, CH)], ssem).wait()

        pltpu.sync_copy(ones_hbm, ones_v)
        pltpu.sync_copy(zeros_hbm, wb0)
        pltpu.sync_copy(zeros_hbm, wb1)
        for t in range(NWB):
            pltpu.async_copy(wbs[t % 2], zslice(t), wsem)
        idx_issue(0)
        idx_issue(1)
        for t in range(NWB):
            pltpu.make_async_copy(wbs[t % 2], zslice(t), wsem).wait()
        plsc.subcore_barrier()

        def body(j, carry):
            @pl.when(j >= 2)
            def _():
                sc_wait()

            @pl.when(j + 2 < NCH)
            def _():
                idx_issue(j + 2)

            idx_wait(j)
            pltpu.async_copy(ones_v, acc.at[idst.at[lax.rem(j, NBI)]],
                             ssem, add=True)
            return carry

        lax.fori_loop(0, NCH, body, 0)
        sc_wait()
        sc_wait()
        plsc.subcore_barrier()
        for t in range(NWB):
            wb = wbs[t % 2]
            if t >= 2:
                pltpu.make_async_copy(wb, oslice(t - 2), wsem).wait()
            pltpu.sync_copy(zslice(t), wb)
            pltpu.async_copy(wb, oslice(t), wsem)
        for t in range(NWB - 2, NWB):
            pltpu.make_async_copy(wbs[t % 2], oslice(t), wsem).wait()

    return k(dst, ones16, zeros16)


def _agg_partials(g, src, dst, zeros80):
    @functools.partial(
        pl.kernel,
        out_type=jax.ShapeDtypeStruct((NC, NP, D), jnp.float32),
        mesh=_sc_mesh(),
        scratch_types=[
            pltpu.VMEM((NBI, CH), jnp.int32),
            pltpu.VMEM((NBI, CH), jnp.int32),
            pltpu.VMEM((NBUF, CH, D), jnp.float32),
            pltpu.VMEM_SHARED((NP, D), jnp.float32),
            pltpu.SemaphoreType.DMA,
            pltpu.SemaphoreType.DMA,
            pltpu.SemaphoreType.DMA,
            pltpu.SemaphoreType.DMA,
        ],
    )
    def k(g_hbm, src_hbm, dst_hbm, zeros_hbm, out_hbm,
          isrc, idst, rows, acc, isem, gsem, ssem, wsem):
        c = lax.axis_index("c")
        s = lax.axis_index("s")
        wid = c * NS + s

        def zslice(t):
            return acc.at[pl.ds(s * RPT + t * CH, CH)]

        def oslice(t):
            return out_hbm.at[c, pl.ds(s * RPT + t * CH, CH)]

        def idx_issue(j):
            base = pl.multiple_of(wid * EPW + j * CH, 8)
            jm = lax.rem(j, NBI)
            pltpu.async_copy(src_hbm.at[pl.ds(base, CH)], isrc.at[jm], isem)
            pltpu.async_copy(dst_hbm.at[pl.ds(base, CH)], idst.at[jm], isem)

        def idx_wait(j):
            base = pl.multiple_of(wid * EPW + j * CH, 8)
            jm = lax.rem(j, NBI)
            pltpu.make_async_copy(src_hbm.at[pl.ds(base, CH)], isrc.at[jm],
                                  isem).wait()
            pltpu.make_async_copy(dst_hbm.at[pl.ds(base, CH)], idst.at[jm],
                                  isem).wait()

        def g_issue(j):
            pltpu.async_copy(g_hbm.at[isrc.at[lax.rem(j, NBI)]],
                             rows.at[lax.rem(j, NBUF)], gsem)

        def g_wait(j):
            pltpu.make_async_copy(g_hbm.at[isrc.at[lax.rem(j, NBI)]],
                                  rows.at[lax.rem(j, NBUF)], gsem).wait()

        def sc_wait():
            pltpu.make_async_copy(rows.at[0], acc.at[pl.ds(0, CH)],
                                  ssem).wait()

        pltpu.sync_copy(zeros_hbm, rows.at[2])
        pltpu.sync_copy(zeros_hbm, rows.at[3])
        for t in range(NWB):
            pltpu.async_copy(rows.at[2 + t % 2], zslice(t), wsem)
        idx_issue(0)
        idx_issue(1)
        idx_wait(0)
        g_issue(0)
        for t in range(NWB):
            pltpu.make_async_copy(rows.at[2 + t % 2], zslice(t), wsem).wait()
        plsc.subcore_barrier()

        def body(j, carry):
            @pl.when(j >= 2)
            def _():
                sc_wait()

            @pl.when(j + 2 < NCH)
            def _():
                idx_issue(j + 2)

            @pl.when(j + 1 < NCH)
            def _():
                idx_wait(j + 1)
                g_issue(j + 1)

            g_wait(j)
            pltpu.async_copy(rows.at[lax.rem(j, NBUF)],
                             acc.at[idst.at[lax.rem(j, NBI)]], ssem, add=True)
            return carry

        lax.fori_loop(0, NCH, body, 0)
        sc_wait()
        sc_wait()
        plsc.subcore_barrier()
        for t in range(NWB):
            wb = rows.at[t % NBUF]
            if t >= NBUF:
                pltpu.make_async_copy(wb, oslice(t - NBUF), wsem).wait()
            pltpu.sync_copy(zslice(t), wb)
            pltpu.async_copy(wb, oslice(t), wsem)
        for t in range(NWB - NBUF, NWB):
            pltpu.make_async_copy(rows.at[t % NBUF], oslice(t), wsem).wait()

    return k(g, src, dst, zeros80)


def _tc_matmul(x, W1):
    def body(x_ref, w_ref, h_ref):
        h_ref[...] = jnp.dot(x_ref[...], w_ref[...],
                             preferred_element_type=jnp.float32)

    return pl.pallas_call(
        body,
        grid=(GRID,),
        in_specs=[
            pl.BlockSpec((BN, D), lambda i: (i, 0)),
            pl.BlockSpec((D, D), lambda i: (0, 0)),
        ],
        out_specs=pl.BlockSpec((BN, D), lambda i: (i, 0)),
        out_shape=jax.ShapeDtypeStruct((N, D), jnp.float32),
    )(x, W1)


def _tc_layer1_pre(h, p):
    def body(h_ref, p_ref, g_ref, dinv_ref):
        deg = p_ref[0, :, 0:8] + p_ref[1, :, 0:8] + 1.0
        dinv = lax.rsqrt(deg)
        dinv_ref[...] = dinv
        g_ref[...] = h_ref[...] * dinv[:, 0:1]

    return pl.pallas_call(
        body,
        grid=(GRID,),
        in_specs=[
            pl.BlockSpec((BN, D), lambda i: (i, 0)),
            pl.BlockSpec((2, BN, DW), lambda i: (0, i, 0)),
        ],
        out_specs=[
            pl.BlockSpec((BN, D), lambda i: (i, 0)),
            pl.BlockSpec((BN, 8), lambda i: (i, 0)),
        ],
        out_shape=[
            jax.ShapeDtypeStruct((N, D), jnp.float32),
            jax.ShapeDtypeStruct((N, 8), jnp.float32),
        ],
    )(h, p)


def _tc_layer1_post(a, g1, dinv, b1r, W2):
    def body(a_ref, g1_ref, dinv_ref, b_ref, w_ref, h1a_ref, g2_ref):
        dv = dinv_ref[...][:, 0:1]
        h1 = (a_ref[0] + a_ref[1] + g1_ref[...]) * dv + b_ref[...]
        h1a = jnp.maximum(h1, 0.0)
        h1a_ref[...] = h1a
        g2_ref[...] = jnp.dot(
            h1a, w_ref[...], preferred_element_type=jnp.float32) * dv

    return pl.pallas_call(
        body,
        grid=(GRID,),
        in_specs=[
            pl.BlockSpec((2, BN, D), lambda i: (0, i, 0)),
            pl.BlockSpec((BN, D), lambda i: (i, 0)),
            pl.BlockSpec((BN, 8), lambda i: (i, 0)),
            pl.BlockSpec((1, D), lambda i: (0, 0)),
            pl.BlockSpec((D, D), lambda i: (0, 0)),
        ],
        out_specs=[
            pl.BlockSpec((BN, D), lambda i: (i, 0)),
            pl.BlockSpec((BN, D), lambda i: (i, 0)),
        ],
        out_shape=[
            jax.ShapeDtypeStruct((N, D), jnp.float32),
            jax.ShapeDtypeStruct((N, D), jnp.float32),
        ],
    )(a, g1, dinv, b1r, W2)


def _tc_layer2_post(a, g2, dinv, b2r):
    def body(a_ref, g2_ref, dinv_ref, b_ref, h2_ref):
        dv = dinv_ref[...][:, 0:1]
        h2_ref[...] = (a_ref[0] + a_ref[1] + g2_ref[...]) * dv + b_ref[...]

    return pl.pallas_call(
        body,
        grid=(GRID,),
        in_specs=[
            pl.BlockSpec((2, BN, D), lambda i: (0, i, 0)),
            pl.BlockSpec((BN, D), lambda i: (i, 0)),
            pl.BlockSpec((BN, 8), lambda i: (i, 0)),
            pl.BlockSpec((1, D), lambda i: (0, 0)),
        ],
        out_specs=pl.BlockSpec((BN, D), lambda i: (i, 0)),
        out_shape=jax.ShapeDtypeStruct((N, D), jnp.float32),
    )(a, g2, dinv, b2r)


def kernel(x, edge_index, W1, b1, W2, b2):
    src = edge_index[0]
    dst = edge_index[1]
    ones16 = jnp.ones((CH, DW), jnp.float32)
    zeros16 = jnp.zeros((CH, DW), jnp.float32)
    zeros80 = jnp.zeros((CH, D), jnp.float32)
    b1r = b1.reshape(1, D)
    b2r = b2.reshape(1, D)

    h1x = _tc_matmul(x, W1)
    degp = _deg_partials(dst, ones16, zeros16).reshape(NC, NP, DW)
    g1, dinv = _tc_layer1_pre(h1x, degp)
    aggp1 = _agg_partials(g1, src, dst, zeros80)
    h1a, g2 = _tc_layer1_post(aggp1, g1, dinv, b1r, W2)
    aggp2 = _agg_partials(g2, src, dst, zeros80)
    h2 = _tc_layer2_post(aggp2, g2, dinv, b2r)
    return (h1a, h2)


# speedup vs baseline: 1.0068x; 1.0068x over previous
import functools

import jax
import jax.numpy as jnp
from jax import lax
from jax.experimental import pallas as pl
from jax.experimental.pallas import tpu as pltpu
from jax.experimental.pallas import tpu_sc as plsc

N = 10000
D = 128
E = 320000
NC = 2
NS = 16
NW = NC * NS
EPW = E // NW
CH = 80
NCH = EPW // CH
NP = 10240
RPT = NP // NS
NWB = RPT // CH
NBUF = 4
NBI = 4

BN = 1000
GRID = N // BN


def _sc_mesh():
    return plsc.VectorSubcoreMesh(
        core_axis_name="c", subcore_axis_name="s",
        num_cores=NC, num_subcores=NS)


DW = 16


def _deg_partials(dst, ones16, zeros16):
    @functools.partial(
        pl.kernel,
        out_type=jax.ShapeDtypeStruct((NC * NP, DW), jnp.float32),
        mesh=_sc_mesh(),
        compiler_params=pltpu.CompilerParams(use_tc_tiling_on_sc=False),
        scratch_types=[
            pltpu.VMEM((NBI, CH), jnp.int32),
            pltpu.VMEM((CH, DW), jnp.float32),
            pltpu.VMEM((CH, DW), jnp.float32),
            pltpu.VMEM((CH, DW), jnp.float32),
            pltpu.VMEM_SHARED((NP, DW), jnp.float32),
            pltpu.SemaphoreType.DMA,
            pltpu.SemaphoreType.DMA,
            pltpu.SemaphoreType.DMA,
        ],
    )
    def k(dst_hbm, ones_hbm, zeros_hbm, out_hbm,
          idst, ones_v, wb0, wb1, acc, isem, ssem, wsem):
        c = lax.axis_index("c")
        s = lax.axis_index("s")
        wid = c * NS + s
        wbs = [wb0, wb1]

        def zslice(t):
            return acc.at[pl.ds(s * RPT + t * CH, CH)]

        def oslice(t):
            return out_hbm.at[pl.ds(c * NP + s * RPT + t * CH, CH)]

        def idx_issue(j):
            base = pl.multiple_of(wid * EPW + j * CH, 8)
            pltpu.async_copy(dst_hbm.at[pl.ds(base, CH)],
                             idst.at[lax.rem(j, NBI)], isem)

        def idx_wait(j):
            base = pl.multiple_of(wid * EPW + j * CH, 8)
            pltpu.make_async_copy(dst_hbm.at[pl.ds(base, CH)],
                                  idst.at[lax.rem(j, NBI)], isem).wait()

        def sc_wait():
            pltpu.make_async_copy(ones_v, acc.at[pl.ds(0, CH)], ssem).wait()

        pltpu.sync_copy(ones_hbm, ones_v)
        pltpu.sync_copy(zeros_hbm, wb0)
        pltpu.sync_copy(zeros_hbm, wb1)
        for t in range(NWB):
            pltpu.async_copy(wbs[t % 2], zslice(t), wsem)
        idx_issue(0)
        idx_issue(1)
        for t in range(NWB):
            pltpu.make_async_copy(wbs[t % 2], zslice(t), wsem).wait()
        plsc.subcore_barrier()

        def body(j, carry):
            @pl.when(j >= 2)
            def _():
                sc_wait()

            @pl.when(j + 2 < NCH)
            def _():
                idx_issue(j + 2)

            idx_wait(j)
            pltpu.async_copy(ones_v, acc.at[idst.at[lax.rem(j, NBI)]],
                             ssem, add=True)
            return carry

        lax.fori_loop(0, NCH, body, 0)
        sc_wait()
        sc_wait()
        plsc.subcore_barrier()
        for t in range(NWB):
            wb = wbs[t % 2]
            if t >= 2:
                pltpu.make_async_copy(wb, oslice(t - 2), wsem).wait()
            pltpu.sync_copy(zslice(t), wb)
            pltpu.async_copy(wb, oslice(t), wsem)
        for t in range(NWB - 2, NWB):
            pltpu.make_async_copy(wbs[t % 2], oslice(t), wsem).wait()

    return k(dst, ones16, zeros16)


def _agg_partials(g, src, dst, zeros80):
    @functools.partial(
        pl.kernel,
        out_type=jax.ShapeDtypeStruct((NC, NP, D), jnp.float32),
        mesh=_sc_mesh(),
        scratch_types=[
            pltpu.VMEM((NBI, CH), jnp.int32),
            pltpu.VMEM((NBI, CH), jnp.int32),
            pltpu.VMEM((NBUF, CH, D), jnp.float32),
            pltpu.VMEM_SHARED((NP, D), jnp.float32),
            pltpu.SemaphoreType.DMA,
            pltpu.SemaphoreType.DMA,
            pltpu.SemaphoreType.DMA,
            pltpu.SemaphoreType.DMA,
        ],
    )
    def k(g_hbm, src_hbm, dst_hbm, zeros_hbm, out_hbm,
          isrc, idst, rows, acc, isem, gsem, ssem, wsem):
        c = lax.axis_index("c")
        s = lax.axis_index("s")
        wid = c * NS + s

        def zslice(t):
            return acc.at[pl.ds(s * RPT + t * CH, CH)]

        def oslice(t):
            return out_hbm.at[c, pl.ds(s * RPT + t * CH, CH)]

        def idx_issue(j):
            base = pl.multiple_of(wid * EPW + j * CH, 8)
            jm = lax.rem(j, NBI)
            pltpu.async_copy(src_hbm.at[pl.ds(base, CH)], isrc.at[jm], isem)
            pltpu.async_copy(dst_hbm.at[pl.ds(base, CH)], idst.at[jm], isem)

        def idx_wait(j):
            base = pl.multiple_of(wid * EPW + j * CH, 8)
            jm = lax.rem(j, NBI)
            pltpu.make_async_copy(src_hbm.at[pl.ds(base, CH)], isrc.at[jm],
                                  isem).wait()
            pltpu.make_async_copy(dst_hbm.at[pl.ds(base, CH)], idst.at[jm],
                                  isem).wait()

        def g_issue(j):
            pltpu.async_copy(g_hbm.at[isrc.at[lax.rem(j, NBI)]],
                             rows.at[lax.rem(j, NBUF)], gsem)

        def g_wait(j):
            pltpu.make_async_copy(g_hbm.at[isrc.at[lax.rem(j, NBI)]],
                                  rows.at[lax.rem(j, NBUF)], gsem).wait()

        def sc_wait():
            pltpu.make_async_copy(rows.at[0], acc.at[pl.ds(0, CH)],
                                  ssem).wait()

        pltpu.sync_copy(zeros_hbm, rows.at[2])
        pltpu.sync_copy(zeros_hbm, rows.at[3])
        for t in range(NWB):
            pltpu.async_copy(rows.at[2 + t % 2], zslice(t), wsem)
        idx_issue(0)
        idx_issue(1)
        idx_wait(0)
        g_issue(0)
        for t in range(NWB):
            pltpu.make_async_copy(rows.at[2 + t % 2], zslice(t), wsem).wait()
        plsc.subcore_barrier()

        def body(j, carry):
            @pl.when(j >= 2)
            def _():
                sc_wait()

            @pl.when(j + 2 < NCH)
            def _():
                idx_issue(j + 2)

            @pl.when(j + 1 < NCH)
            def _():
                idx_wait(j + 1)
                g_issue(j + 1)

            g_wait(j)
            pltpu.async_copy(rows.at[lax.rem(j, NBUF)],
                             acc.at[idst.at[lax.rem(j, NBI)]], ssem, add=True)
            return carry

        lax.fori_loop(0, NCH, body, 0)
        sc_wait()
        sc_wait()
        plsc.subcore_barrier()
        for t in range(NWB):
            wb = rows.at[t % NBUF]
            if t >= NBUF:
                pltpu.make_async_copy(wb, oslice(t - NBUF), wsem).wait()
            pltpu.sync_copy(zslice(t), wb)
            pltpu.async_copy(wb, oslice(t), wsem)
        for t in range(NWB - NBUF, NWB):
            pltpu.make_async_copy(rows.at[t % NBUF], oslice(t), wsem).wait()

    return k(g, src, dst, zeros80)


def _tc_layer1_pre(x, W1, p):
    def body(x_ref, w_ref, p_ref, g_ref, dinv_ref):
        deg = p_ref[0, :, 0:8] + p_ref[1, :, 0:8] + 1.0
        dinv = lax.rsqrt(deg)
        dinv_ref[...] = dinv
        h = jnp.dot(x_ref[...], w_ref[...], preferred_element_type=jnp.float32)
        g_ref[...] = h * dinv[:, 0:1]

    return pl.pallas_call(
        body,
        grid=(GRID,),
        in_specs=[
            pl.BlockSpec((BN, D), lambda i: (i, 0)),
            pl.BlockSpec((D, D), lambda i: (0, 0)),
            pl.BlockSpec((2, BN, DW), lambda i: (0, i, 0)),
        ],
        out_specs=[
            pl.BlockSpec((BN, D), lambda i: (i, 0)),
            pl.BlockSpec((BN, 8), lambda i: (i, 0)),
        ],
        out_shape=[
            jax.ShapeDtypeStruct((N, D), jnp.float32),
            jax.ShapeDtypeStruct((N, 8), jnp.float32),
        ],
    )(x, W1, p)


def _tc_layer1_post(a, g1, dinv, b1r, W2):
    def body(a_ref, g1_ref, dinv_ref, b_ref, w_ref, h1a_ref, g2_ref):
        dv = dinv_ref[...][:, 0:1]
        h1 = (a_ref[0] + a_ref[1] + g1_ref[...]) * dv + b_ref[...]
        h1a = jnp.maximum(h1, 0.0)
        h1a_ref[...] = h1a
        g2_ref[...] = jnp.dot(
            h1a, w_ref[...], preferred_element_type=jnp.float32) * dv

    return pl.pallas_call(
        body,
        grid=(GRID,),
        in_specs=[
            pl.BlockSpec((2, BN, D), lambda i: (0, i, 0)),
            pl.BlockSpec((BN, D), lambda i: (i, 0)),
            pl.BlockSpec((BN, 8), lambda i: (i, 0)),
            pl.BlockSpec((1, D), lambda i: (0, 0)),
            pl.BlockSpec((D, D), lambda i: (0, 0)),
        ],
        out_specs=[
            pl.BlockSpec((BN, D), lambda i: (i, 0)),
            pl.BlockSpec((BN, D), lambda i: (i, 0)),
        ],
        out_shape=[
            jax.ShapeDtypeStruct((N, D), jnp.float32),
            jax.ShapeDtypeStruct((N, D), jnp.float32),
        ],
    )(a, g1, dinv, b1r, W2)


def _tc_layer2_post(a, g2, dinv, b2r):
    def body(a_ref, g2_ref, dinv_ref, b_ref, h2_ref):
        dv = dinv_ref[...][:, 0:1]
        h2_ref[...] = (a_ref[0] + a_ref[1] + g2_ref[...]) * dv + b_ref[...]

    return pl.pallas_call(
        body,
        grid=(GRID,),
        in_specs=[
            pl.BlockSpec((2, BN, D), lambda i: (0, i, 0)),
            pl.BlockSpec((BN, D), lambda i: (i, 0)),
            pl.BlockSpec((BN, 8), lambda i: (i, 0)),
            pl.BlockSpec((1, D), lambda i: (0, 0)),
        ],
        out_specs=pl.BlockSpec((BN, D), lambda i: (i, 0)),
        out_shape=jax.ShapeDtypeStruct((N, D), jnp.float32),
    )(a, g2, dinv, b2r)


def kernel(x, edge_index, W1, b1, W2, b2):
    src = edge_index[0]
    dst = edge_index[1]
    ones16 = jnp.ones((CH, DW), jnp.float32)
    zeros16 = jnp.zeros((CH, DW), jnp.float32)
    zeros80 = jnp.zeros((CH, D), jnp.float32)
    b1r = b1.reshape(1, D)
    b2r = b2.reshape(1, D)

    degp = _deg_partials(dst, ones16, zeros16).reshape(NC, NP, DW)
    g1, dinv = _tc_layer1_pre(x, W1, degp)
    aggp1 = _agg_partials(g1, src, dst, zeros80)
    h1a, g2 = _tc_layer1_post(aggp1, g1, dinv, b1r, W2)
    aggp2 = _agg_partials(g2, src, dst, zeros80)
    h2 = _tc_layer2_post(aggp2, g2, dinv, b2r)
    return (h1a, h2)
